# baseline (speedup 1.0000x reference)
import functools

import jax
import jax.numpy as jnp
from jax import lax
from jax.experimental import pallas as pl
from jax.experimental.pallas import tpu as pltpu
from jax.experimental.pallas import tpu_sc as plsc

_INFO = plsc.get_sparse_core_info()
_NC = _INFO.num_cores
_NS = _INFO.num_subcores
_NW = _NC * _NS
_L = _INFO.num_lanes

_B = 16384
_J = 16
_K = 64
_BPW = _B // _NW
_EPW = _BPW * _J
_ROWS = _EPW // 128
_NG = _BPW // _L


def _sc_gather(xflat, idx0, idx2):
    mesh = plsc.VectorSubcoreMesh(core_axis_name="c", subcore_axis_name="s")

    @functools.partial(
        pl.kernel,
        out_type=jax.ShapeDtypeStruct((_B * _J,), jnp.float32),
        mesh=mesh,
        compiler_params=pltpu.CompilerParams(needs_layout_passes=False),
        scratch_types=[
            pltpu.VMEM((_BPW,), jnp.int32),
            pltpu.VMEM((_BPW,), jnp.int32),
            pltpu.VMEM((_EPW,), jnp.int32),
            pltpu.VMEM((_EPW,), jnp.float32),
            pltpu.SemaphoreType.DMA,
        ],
    )
    def k(x_hbm, idx0_hbm, idx2_hbm, out_hbm, i0_v, i2_v, idx_v, out_v, sem):
        wid = lax.axis_index("s") * _NC + lax.axis_index("c")
        base_b = wid * _BPW
        pltpu.sync_copy(idx0_hbm.at[pl.ds(base_b, _BPW)], i0_v)
        pltpu.sync_copy(idx2_hbm.at[pl.ds(base_b, _BPW)], i2_v)

        lane = lax.iota(jnp.int32, _L)

        def gbody(g, carry):
            i0 = i0_v[pl.ds(g * _L, _L)]
            i2 = i2_v[pl.ds(g * _L, _L)]
            base2 = i0 * (_J * _K) + i2
            pb = g * 256 + lane * _J
            for j in range(_J):
                plsc.store_scatter(idx_v, [pb + j], base2 + j * _K)
            pltpu.async_copy(
                x_hbm.at[idx_v.at[pl.ds(g * 256, 128)]],
                out_v.at[pl.ds(g * 256, 128)], sem)
            pltpu.async_copy(
                x_hbm.at[idx_v.at[pl.ds(g * 256 + 128, 128)]],
                out_v.at[pl.ds(g * 256 + 128, 128)], sem)
            return carry

        lax.fori_loop(0, _NG, gbody, 0)
        pltpu.make_async_copy(x_hbm.at[pl.ds(0, _EPW)], out_v, sem).wait()
        pltpu.sync_copy(out_v, out_hbm.at[pl.ds(base_b * _J, _EPW)])

    return k(xflat, idx0, idx2)


def kernel(x, idx0, idx2):
    xflat = x.reshape(-1)
    out = _sc_gather(xflat, idx0.astype(jnp.int32), idx2.astype(jnp.int32))
    return out.reshape(_B, _J)


# speedup vs baseline: 1.5471x; 1.5471x over previous
import functools

import jax
import jax.numpy as jnp
from jax import lax
from jax.experimental import pallas as pl
from jax.experimental.pallas import tpu as pltpu
from jax.experimental.pallas import tpu_sc as plsc

_INFO = plsc.get_sparse_core_info()
_NC = _INFO.num_cores
_NS = _INFO.num_subcores
_NW = _NC * _NS
_L = _INFO.num_lanes

_B = 16384
_J = 16
_K = 64
_NI = 100000
_BPW = _B // _NW
_EPW = _BPW * _J
_ROWS = _EPW // 128
_NG = _BPW // _L


def _sc_gather(xflat, idx0, idx2):
    mesh = plsc.VectorSubcoreMesh(core_axis_name="c", subcore_axis_name="s")

    @functools.partial(
        pl.kernel,
        out_type=jax.ShapeDtypeStruct((_B * _J,), jnp.float32),
        mesh=mesh,
        compiler_params=pltpu.CompilerParams(needs_layout_passes=False),
        scratch_types=[
            pltpu.VMEM((_BPW,), jnp.int32),
            pltpu.VMEM((_BPW,), jnp.int32),
            pltpu.VMEM((_EPW,), jnp.int32),
            pltpu.VMEM((_EPW,), jnp.float32),
            pltpu.SemaphoreType.DMA,
        ],
    )
    def k(x_hbm, idx0_hbm, idx2_hbm, out_hbm, i0_v, i2_v, idx_v, out_v, sem):
        wid = lax.axis_index("s") * _NC + lax.axis_index("c")
        base_b = wid * _BPW
        pltpu.sync_copy(idx0_hbm.at[pl.ds(base_b, _BPW)], i0_v)
        pltpu.sync_copy(idx2_hbm.at[pl.ds(base_b, _BPW)], i2_v)

        lane = lax.iota(jnp.int32, _L)

        def gbody(g, carry):
            i0 = i0_v[pl.ds(g * _L, _L)]
            i2 = i2_v[pl.ds(g * _L, _L)]
            base2 = i2 * _NI + i0
            pb = g * 256 + lane * _J
            for j in range(_J):
                plsc.store_scatter(idx_v, [pb + j], base2 + j * (_K * _NI))
            pltpu.async_copy(
                x_hbm.at[idx_v.at[pl.ds(g * 256, 128)]],
                out_v.at[pl.ds(g * 256, 128)], sem)
            pltpu.async_copy(
                x_hbm.at[idx_v.at[pl.ds(g * 256 + 128, 128)]],
                out_v.at[pl.ds(g * 256 + 128, 128)], sem)
            return carry

        lax.fori_loop(0, _NG, gbody, 0)
        pltpu.make_async_copy(x_hbm.at[pl.ds(0, _EPW)], out_v, sem).wait()
        pltpu.sync_copy(out_v, out_hbm.at[pl.ds(base_b * _J, _EPW)])

    return k(xflat, idx0, idx2)


def kernel(x, idx0, idx2):
    xflat = x.transpose(1, 2, 0).reshape(-1)
    out = _sc_gather(xflat, idx0.astype(jnp.int32), idx2.astype(jnp.int32))
    return out.reshape(_B, _J)


# speedup vs baseline: 2.9630x; 1.9152x over previous
import functools

import jax
import jax.numpy as jnp
from jax import lax
from jax.experimental import pallas as pl
from jax.experimental.pallas import tpu as pltpu
from jax.experimental.pallas import tpu_sc as plsc

_INFO = plsc.get_sparse_core_info()
_NC = _INFO.num_cores
_NS = _INFO.num_subcores
_NW = _NC * _NS
_L = _INFO.num_lanes

_B = 16384
_J = 16
_K = 64
_NI = 100000
_BPW = _B // _NW
_LAG = 0


def _sc_gather(xt, idx0, idx2):
    mesh = plsc.VectorSubcoreMesh(core_axis_name="c", subcore_axis_name="s")

    @functools.partial(
        pl.kernel,
        out_type=jax.ShapeDtypeStruct((_B * _J,), jnp.float32),
        mesh=mesh,
        compiler_params=pltpu.CompilerParams(needs_layout_passes=False,
                                             disable_bounds_checks=True),
        scratch_types=[
            pltpu.VMEM((_BPW,), jnp.int32),
            pltpu.VMEM((_BPW,), jnp.int32),
            pltpu.VMEM((_LAG + 1, _J, 1, 1, 128), jnp.float32),
            pltpu.VMEM((_BPW * _J,), jnp.float32),
            pltpu.VMEM((_J * 16,), jnp.int32),
            pltpu.SemaphoreType.DMA,
        ],
    )
    def k(x_hbm, idx0_hbm, idx2_hbm, out_hbm, i0_v, i2_v,
          slab_v, out_v, dummy_v, sem):
        wid = lax.axis_index("s") * _NC + lax.axis_index("c")
        base_b = wid * _BPW
        pltpu.sync_copy(idx0_hbm.at[pl.ds(base_b, _BPW)], i0_v)
        pltpu.sync_copy(idx2_hbm.at[pl.ds(base_b, _BPW)], i2_v)

        jlane = lax.iota(jnp.int32, _L)

        def scalar_at(ref, b):
            vec = ref[pl.ds((b >> 4) * _L, _L)]
            masked = jnp.where(jlane == (b & 15), vec, 0)
            return lax.reduce_max(masked, axes=(0,))

        def body(b, carry):
            @pl.when(b < _BPW)
            def _fire():
                i = scalar_at(i0_v, b)
                kk = scalar_at(i2_v, b)
                q = kk >> 3
                poff = ((i >> 7) << 10) + ((kk & 7) << 7) + (((i >> 4) & 7) << 4)
                col0 = pl.multiple_of(poff, 128)
                for j in range(_J):
                    pltpu.async_copy(
                        x_hbm.at[pl.ds(8 * j + q, 1), pl.ds(0, 1),
                                 pl.ds(col0, 16)],
                        slab_v.at[b & _LAG, j, pl.ds(0, 1), pl.ds(0, 1),
                                  pl.ds(0, 16)], sem)

            @pl.when(b >= _LAG)
            def _select():
                bl = b - _LAG
                pltpu.make_async_copy(
                    idx0_hbm.at[pl.ds(0, _J * 16)], dummy_v, sem).wait()
                i = scalar_at(i0_v, bl)
                zero = jnp.full((_L,), 0, jnp.int32)
                vals = plsc.load_gather(
                    slab_v,
                    [zero + (bl & _LAG),
                     jlane,
                     zero,
                     zero,
                     zero + (i & 15)])
                out_v[pl.ds(bl * _J, _L)] = vals
            return carry

        lax.fori_loop(0, _BPW + _LAG, body, 0)
        pltpu.sync_copy(out_v, out_hbm.at[pl.ds(base_b * _J, _BPW * _J)])

    return k(xt, idx0, idx2)


def kernel(x, idx0, idx2):
    xt = x.transpose(1, 2, 0).reshape(128, 8, _NI)
    out = _sc_gather(xt, idx0.astype(jnp.int32), idx2.astype(jnp.int32))
    return out.reshape(_B, _J)


# speedup vs baseline: 9.4076x; 3.1751x over previous
import functools

import jax
import jax.numpy as jnp
from jax import lax
from jax.experimental import pallas as pl
from jax.experimental.pallas import tpu as pltpu
from jax.experimental.pallas import tpu_sc as plsc

_INFO = plsc.get_sparse_core_info()
_NC = _INFO.num_cores
_NS = _INFO.num_subcores
_NW = _NC * _NS
_L = _INFO.num_lanes

_B = 16384
_J = 16
_K = 64
_NI = 100000
_BPW = _B // _NW
_LAG = 7


def _sc_gather(xt, idx0, idx2):
    mesh = plsc.VectorSubcoreMesh(core_axis_name="c", subcore_axis_name="s")

    @functools.partial(
        pl.kernel,
        out_type=jax.ShapeDtypeStruct((_B * _J,), jnp.float32),
        mesh=mesh,
        compiler_params=pltpu.CompilerParams(needs_layout_passes=False,
                                             disable_bounds_checks=True),
        scratch_types=[
            pltpu.VMEM((_BPW,), jnp.int32),
            pltpu.VMEM((_BPW,), jnp.int32),
            pltpu.VMEM((_LAG + 1, _J, 1, 1, 128), jnp.float32),
            pltpu.VMEM((_BPW * _J,), jnp.float32),
            pltpu.VMEM((_J * 16,), jnp.int32),
            pltpu.SemaphoreType.DMA,
        ],
    )
    def k(x_hbm, idx0_hbm, idx2_hbm, out_hbm, i0_v, i2_v,
          slab_v, out_v, dummy_v, sem):
        wid = lax.axis_index("s") * _NC + lax.axis_index("c")
        base_b = wid * _BPW
        pltpu.sync_copy(idx0_hbm.at[pl.ds(base_b, _BPW)], i0_v)
        pltpu.sync_copy(idx2_hbm.at[pl.ds(base_b, _BPW)], i2_v)

        jlane = lax.iota(jnp.int32, _L)

        def scalar_at(ref, b):
            vec = ref[pl.ds((b >> 4) * _L, _L)]
            masked = jnp.where(jlane == (b & 15), vec, 0)
            return lax.reduce_max(masked, axes=(0,))

        def body(b, carry):
            @pl.when(b < _BPW)
            def _fire():
                i = scalar_at(i0_v, b)
                kk = scalar_at(i2_v, b)
                q = kk >> 3
                poff = ((i >> 7) << 10) + ((kk & 7) << 7) + (((i >> 4) & 7) << 4)
                col0 = pl.multiple_of(poff, 128)
                for j in range(_J):
                    pltpu.async_copy(
                        x_hbm.at[pl.ds(8 * j + q, 1), pl.ds(0, 1),
                                 pl.ds(col0, 16)],
                        slab_v.at[b & _LAG, j, pl.ds(0, 1), pl.ds(0, 1),
                                  pl.ds(0, 16)], sem)

            @pl.when(b >= _LAG)
            def _select():
                bl = b - _LAG
                pltpu.make_async_copy(
                    idx0_hbm.at[pl.ds(0, _J * 16)], dummy_v, sem).wait()
                i = scalar_at(i0_v, bl)
                zero = jnp.full((_L,), 0, jnp.int32)
                vals = plsc.load_gather(
                    slab_v,
                    [zero + (bl & _LAG),
                     jlane,
                     zero,
                     zero,
                     zero + (i & 15)])
                out_v[pl.ds(bl * _J, _L)] = vals
            return carry

        lax.fori_loop(0, _BPW + _LAG, body, 0)
        pltpu.sync_copy(out_v, out_hbm.at[pl.ds(base_b * _J, _BPW * _J)])

    return k(xt, idx0, idx2)


def kernel(x, idx0, idx2):
    xt = x.transpose(1, 2, 0).reshape(128, 8, _NI)
    out = _sc_gather(xt, idx0.astype(jnp.int32), idx2.astype(jnp.int32))
    return out.reshape(_B, _J)


# speedup vs baseline: 11.3238x; 1.2037x over previous
import functools

import jax
import jax.numpy as jnp
from jax import lax
from jax.experimental import pallas as pl
from jax.experimental.pallas import tpu as pltpu
from jax.experimental.pallas import tpu_sc as plsc

_INFO = plsc.get_sparse_core_info()
_NC = _INFO.num_cores
_NS = _INFO.num_subcores
_NW = _NC * _NS
_L = _INFO.num_lanes

_B = 16384
_J = 16
_K = 64
_NI = 100000
_BPW = _B // _NW
_LAG = 7


def _sc_gather(xt, idx0, idx2):
    mesh = plsc.VectorSubcoreMesh(core_axis_name="c", subcore_axis_name="s")

    @functools.partial(
        pl.kernel,
        out_type=jax.ShapeDtypeStruct((_B * _J,), jnp.float32),
        mesh=mesh,
        compiler_params=pltpu.CompilerParams(needs_layout_passes=False,
                                             disable_bounds_checks=True),
        scratch_types=[
            pltpu.VMEM((_BPW,), jnp.int32),
            pltpu.VMEM((_BPW,), jnp.int32),
            pltpu.VMEM((_LAG + 1, _J, 1, 1, 128), jnp.float32),
            pltpu.VMEM((_BPW * _J,), jnp.float32),
            pltpu.VMEM((_J * 16,), jnp.int32),
            pltpu.SemaphoreType.DMA((_LAG + 1,)),
        ],
    )
    def k(x_hbm, idx0_hbm, idx2_hbm, out_hbm, i0_v, i2_v,
          slab_v, out_v, dummy_v, sem):
        wid = lax.axis_index("s") * _NC + lax.axis_index("c")
        base_b = wid * _BPW
        pltpu.sync_copy(idx0_hbm.at[pl.ds(base_b, _BPW)], i0_v)
        pltpu.sync_copy(idx2_hbm.at[pl.ds(base_b, _BPW)], i2_v)

        jlane = lax.iota(jnp.int32, _L)

        def scalar_at(ref, b):
            vec = ref[pl.ds((b >> 4) * _L, _L)]
            masked = jnp.where(jlane == (b & 15), vec, 0)
            return lax.reduce_max(masked, axes=(0,))

        def body(b, carry):
            @pl.when(b < _BPW)
            def _fire():
                i = scalar_at(i0_v, b)
                kk = scalar_at(i2_v, b)
                q = kk >> 3
                poff = ((i >> 7) << 10) + ((kk & 7) << 7) + (((i >> 4) & 7) << 4)
                col0 = pl.multiple_of(poff, 128)
                for j in range(_J):
                    pltpu.async_copy(
                        x_hbm.at[pl.ds(8 * j + q, 1), pl.ds(0, 1),
                                 pl.ds(col0, 16)],
                        slab_v.at[b & _LAG, j, pl.ds(0, 1), pl.ds(0, 1),
                                  pl.ds(0, 16)], sem.at[b & _LAG])

            @pl.when(b >= _LAG)
            def _select():
                bl = b - _LAG
                pltpu.make_async_copy(
                    idx0_hbm.at[pl.ds(0, _J * 16)], dummy_v,
                    sem.at[bl & _LAG]).wait()
                i = scalar_at(i0_v, bl)
                zero = jnp.full((_L,), 0, jnp.int32)
                vals = plsc.load_gather(
                    slab_v,
                    [zero + (bl & _LAG),
                     jlane,
                     zero,
                     zero,
                     zero + (i & 15)])
                out_v[pl.ds(bl * _J, _L)] = vals
            return carry

        lax.fori_loop(0, _BPW + _LAG, body, 0)
        pltpu.sync_copy(out_v, out_hbm.at[pl.ds(base_b * _J, _BPW * _J)])

    return k(xt, idx0, idx2)


def kernel(x, idx0, idx2):
    xt = x.transpose(1, 2, 0).reshape(128, 8, _NI)
    out = _sc_gather(xt, idx0.astype(jnp.int32), idx2.astype(jnp.int32))
    return out.reshape(_B, _J)


# speedup vs baseline: 12.3201x; 1.0880x over previous
import functools

import jax
import jax.numpy as jnp
from jax import lax
from jax.experimental import pallas as pl
from jax.experimental.pallas import tpu as pltpu
from jax.experimental.pallas import tpu_sc as plsc

_INFO = plsc.get_sparse_core_info()
_NC = _INFO.num_cores
_NS = _INFO.num_subcores
_NW = _NC * _NS
_L = _INFO.num_lanes

_B = 16384
_J = 16
_K = 64
_NI = 100000
_BPW = _B // _NW
_LAG = 7


def _sc_gather(xt, idx0, idx2):
    mesh = plsc.VectorSubcoreMesh(core_axis_name="c", subcore_axis_name="s")

    @functools.partial(
        pl.kernel,
        out_type=jax.ShapeDtypeStruct((_B * _J,), jnp.float32),
        mesh=mesh,
        compiler_params=pltpu.CompilerParams(needs_layout_passes=False,
                                             disable_bounds_checks=True),
        scratch_types=[
            pltpu.VMEM((_BPW,), jnp.int32),
            pltpu.VMEM((_BPW,), jnp.int32),
            pltpu.VMEM((_LAG + 1, _J, 1, 128), jnp.float32),
            pltpu.VMEM((_BPW * _J,), jnp.float32),
            pltpu.VMEM((_J * 16,), jnp.int32),
            pltpu.SemaphoreType.DMA((_LAG + 1,)),
        ],
    )
    def k(x_hbm, idx0_hbm, idx2_hbm, out_hbm, i0_v, i2_v,
          slab_v, out_v, dummy_v, sem):
        wid = lax.axis_index("s") * _NC + lax.axis_index("c")
        base_b = wid * _BPW
        pltpu.sync_copy(idx0_hbm.at[pl.ds(base_b, _BPW)], i0_v)
        pltpu.sync_copy(idx2_hbm.at[pl.ds(base_b, _BPW)], i2_v)

        jlane = lax.iota(jnp.int32, _L)

        def scalar_at(ref, b):
            vec = ref[pl.ds((b >> 4) * _L, _L)]
            masked = jnp.where(jlane == (b & 15), vec, 0)
            return lax.reduce_max(masked, axes=(0,))

        def body(b, carry):
            @pl.when(b < _BPW)
            def _fire():
                i = scalar_at(i0_v, b)
                kk = scalar_at(i2_v, b)
                poff = ((kk >> 3) * 800768 + ((i >> 7) << 10)
                        + ((kk & 7) << 7) + (((i >> 4) & 7) << 4))
                col0 = pl.multiple_of(poff, 128)
                pltpu.async_copy(
                    x_hbm.at[pl.ds(0, _J), pl.ds(0, 1), pl.ds(col0, 16)],
                    slab_v.at[b & _LAG, pl.ds(0, _J), pl.ds(0, 1),
                              pl.ds(0, 16)], sem.at[b & _LAG])

            @pl.when(b >= _LAG)
            def _select():
                bl = b - _LAG
                pltpu.make_async_copy(
                    idx0_hbm.at[pl.ds(0, _J * 16)], dummy_v,
                    sem.at[bl & _LAG]).wait()
                i = scalar_at(i0_v, bl)
                zero = jnp.full((_L,), 0, jnp.int32)
                vals = plsc.load_gather(
                    slab_v,
                    [zero + (bl & _LAG),
                     jlane,
                     zero,
                     zero + (i & 15)])
                out_v[pl.ds(bl * _J, _L)] = vals
            return carry

        lax.fori_loop(0, _BPW + _LAG, body, 0)
        pltpu.sync_copy(out_v, out_hbm.at[pl.ds(base_b * _J, _BPW * _J)])

    return k(xt, idx0, idx2)


def kernel(x, idx0, idx2):
    xt = x.transpose(1, 2, 0)
    out = _sc_gather(xt, idx0.astype(jnp.int32), idx2.astype(jnp.int32))
    return out.reshape(_B, _J)


# speedup vs baseline: 12.4217x; 1.0082x over previous
import functools

import jax
import jax.numpy as jnp
from jax import lax
from jax.experimental import pallas as pl
from jax.experimental.pallas import tpu as pltpu
from jax.experimental.pallas import tpu_sc as plsc

_INFO = plsc.get_sparse_core_info()
_NC = _INFO.num_cores
_NS = _INFO.num_subcores
_NW = _NC * _NS
_L = _INFO.num_lanes

_B = 16384
_J = 16
_K = 64
_NI = 100000
_BPW = _B // _NW
_LAG = 15


def _sc_gather(xt, idx0, idx2):
    mesh = plsc.VectorSubcoreMesh(core_axis_name="c", subcore_axis_name="s")

    @functools.partial(
        pl.kernel,
        out_type=jax.ShapeDtypeStruct((_B * _J,), jnp.float32),
        mesh=mesh,
        compiler_params=pltpu.CompilerParams(needs_layout_passes=False,
                                             disable_bounds_checks=True),
        scratch_types=[
            pltpu.VMEM((_BPW,), jnp.int32),
            pltpu.VMEM((_BPW,), jnp.int32),
            pltpu.VMEM((_LAG + 1, _J, 1, 128), jnp.float32),
            pltpu.VMEM((_BPW * _J,), jnp.float32),
            pltpu.VMEM((_J * 16,), jnp.int32),
            pltpu.SemaphoreType.DMA((_LAG + 1,)),
        ],
    )
    def k(x_hbm, idx0_hbm, idx2_hbm, out_hbm, i0_v, i2_v,
          slab_v, out_v, dummy_v, sem):
        wid = lax.axis_index("s") * _NC + lax.axis_index("c")
        base_b = wid * _BPW
        pltpu.sync_copy(idx0_hbm.at[pl.ds(base_b, _BPW)], i0_v)
        pltpu.sync_copy(idx2_hbm.at[pl.ds(base_b, _BPW)], i2_v)

        jlane = lax.iota(jnp.int32, _L)

        def scalar_at(ref, b):
            vec = ref[pl.ds((b >> 4) * _L, _L)]
            masked = jnp.where(jlane == (b & 15), vec, 0)
            return lax.reduce_max(masked, axes=(0,))

        def body(b, carry):
            @pl.when(b < _BPW)
            def _fire():
                i = scalar_at(i0_v, b)
                kk = scalar_at(i2_v, b)
                poff = ((kk >> 3) * 800768 + ((i >> 7) << 10)
                        + ((kk & 7) << 7) + (((i >> 4) & 7) << 4))
                col0 = pl.multiple_of(poff, 128)
                pltpu.async_copy(
                    x_hbm.at[pl.ds(0, _J), pl.ds(0, 1), pl.ds(col0, 16)],
                    slab_v.at[b & _LAG, pl.ds(0, _J), pl.ds(0, 1),
                              pl.ds(0, 16)], sem.at[b & _LAG])

            @pl.when(b >= _LAG)
            def _select():
                bl = b - _LAG
                pltpu.make_async_copy(
                    idx0_hbm.at[pl.ds(0, _J * 16)], dummy_v,
                    sem.at[bl & _LAG]).wait()
                i = scalar_at(i0_v, bl)
                zero = jnp.full((_L,), 0, jnp.int32)
                vals = plsc.load_gather(
                    slab_v,
                    [zero + (bl & _LAG),
                     jlane,
                     zero,
                     zero + (i & 15)])
                out_v[pl.ds(bl * _J, _L)] = vals
            return carry

        lax.fori_loop(0, _BPW + _LAG, body, 0)
        pltpu.sync_copy(out_v, out_hbm.at[pl.ds(base_b * _J, _BPW * _J)])

    return k(xt, idx0, idx2)


def kernel(x, idx0, idx2):
    xt = x.transpose(1, 2, 0)
    out = _sc_gather(xt, idx0.astype(jnp.int32), idx2.astype(jnp.int32))
    return out.reshape(_B, _J)


# speedup vs baseline: 15.4141x; 1.2409x over previous
import functools

import jax
import jax.numpy as jnp
from jax import lax
from jax.experimental import pallas as pl
from jax.experimental.pallas import tpu as pltpu
from jax.experimental.pallas import tpu_sc as plsc

_INFO = plsc.get_sparse_core_info()
_NC = _INFO.num_cores
_NS = _INFO.num_subcores
_NW = _NC * _NS
_L = _INFO.num_lanes

_B = 16384
_J = 16
_K = 64
_NI = 100000
_BPW = _B // _NW
_NG = _BPW // _L
_RING = 32


def _sc_gather(xt, idx0, idx2):
    mesh = plsc.VectorSubcoreMesh(core_axis_name="c", subcore_axis_name="s")

    @functools.partial(
        pl.kernel,
        out_type=jax.ShapeDtypeStruct((_B * _J,), jnp.float32),
        mesh=mesh,
        compiler_params=pltpu.CompilerParams(needs_layout_passes=False,
                                             disable_bounds_checks=True),
        scratch_types=[
            pltpu.VMEM((_BPW,), jnp.int32),
            pltpu.VMEM((_BPW,), jnp.int32),
            pltpu.VMEM((_BPW,), jnp.int32),
            pltpu.VMEM((_RING, _J, 1, 128), jnp.float32),
            pltpu.VMEM((_BPW * _J,), jnp.float32),
            pltpu.VMEM((_L * 256,), jnp.int32),
            pltpu.SemaphoreType.DMA((2,)),
        ],
    )
    def k(x_hbm, idx0_hbm, idx2_hbm, out_hbm, i0_v, i2_v, poff_v,
          slab_v, out_v, dummy_v, sem):
        wid = lax.axis_index("s") * _NC + lax.axis_index("c")
        base_b = wid * _BPW
        pltpu.sync_copy(idx0_hbm.at[pl.ds(base_b, _BPW)], i0_v)
        pltpu.sync_copy(idx2_hbm.at[pl.ds(base_b, _BPW)], i2_v)

        jlane = lax.iota(jnp.int32, _L)

        def pbody(g, carry):
            i = i0_v[pl.ds(g * _L, _L)]
            kk = i2_v[pl.ds(g * _L, _L)]
            poff_v[pl.ds(g * _L, _L)] = ((kk >> 3) * 800768
                                         + ((i >> 7) << 10)
                                         + ((kk & 7) << 7)
                                         + (((i >> 4) & 7) << 4))
            return carry

        lax.fori_loop(0, _NG, pbody, 0)

        def scalar_at(ref, b):
            vec = ref[pl.ds((b >> 4) * _L, _L)]
            masked = jnp.where(jlane == (b & 15), vec, 0)
            return lax.reduce_max(masked, axes=(0,))

        def body(b, carry):
            @pl.when(b < _BPW)
            def _fire():
                col0 = pl.multiple_of(scalar_at(poff_v, b), 128)
                pltpu.async_copy(
                    x_hbm.at[pl.ds(0, _J), pl.ds(0, 1), pl.ds(col0, 16)],
                    slab_v.at[b % _RING, pl.ds(0, _J), pl.ds(0, 1),
                              pl.ds(0, 16)], sem.at[(b >> 4) & 1])

            @pl.when(jnp.logical_and((b & 15) == 15, b >= (_RING - 1)))
            def _select():
                g = (b >> 4) - 1
                pltpu.make_async_copy(
                    idx0_hbm.at[pl.ds(0, _L * 256)], dummy_v,
                    sem.at[g & 1]).wait()
                lane_sel = i0_v[pl.ds(g * _L, _L)] & 15
                slot0 = (g & 1) * _L
                zero = jnp.full((_L,), 0, jnp.int32)
                obase = g * (_L * _J)
                for j in range(_J):
                    vals = plsc.load_gather(
                        slab_v,
                        [zero + slot0 + jlane,
                         zero + j,
                         zero,
                         lane_sel])
                    plsc.store_scatter(out_v, [obase + jlane * _J + j], vals)
            return carry

        lax.fori_loop(0, _BPW + _L, body, 0)
        pltpu.sync_copy(out_v, out_hbm.at[pl.ds(base_b * _J, _BPW * _J)])

    return k(xt, idx0, idx2)


def kernel(x, idx0, idx2):
    xt = x.transpose(1, 2, 0)
    out = _sc_gather(xt, idx0.astype(jnp.int32), idx2.astype(jnp.int32))
    return out.reshape(_B, _J)
